# baseline (device time: 18671 ns/iter reference)
import jax
import jax.numpy as jnp
from jax import lax
from jax.experimental import pallas as pl
from jax.experimental.pallas import tpu as pltpu

N_DEV = 16
WIN = 128


def kernel(x, Wq, K_ext, V_ext, Wo):
    B, Sq, Dm = x.shape
    _, Skv, Hq, Dh = K_ext.shape
    D = Hq * Dh
    Se = Skv + 2 * WIN
    NQB = Sq // WIN
    KB = 3 * WIN

    def body(x_ref, wq_ref, k_ref, v_ref, wo_ref, out_ref,
             kbuf, vbuf, send_sems, recv_sems):
        me = lax.axis_index("i")
        left = lax.rem(me - 1 + N_DEV, N_DEV)
        right = lax.rem(me + 1, N_DEV)

        barrier_sem = pltpu.get_barrier_semaphore()
        for nbr in (left, right):
            pl.semaphore_signal(barrier_sem, inc=1, device_id=(nbr,),
                                device_id_type=pl.DeviceIdType.MESH)
        pl.semaphore_wait(barrier_sem, 2)

        kbuf[:, WIN:WIN + Skv, :] = (
            k_ref[...].reshape(B, Skv, D).astype(jnp.bfloat16))
        vbuf[:, WIN:WIN + Skv, :] = (
            v_ref[...].reshape(B, Skv, D).astype(jnp.bfloat16))

        plan = [
            (kbuf, Skv, 0, right, 0),
            (vbuf, Skv, 0, right, 1),
            (kbuf, WIN, WIN + Skv, left, 2),
            (vbuf, WIN, WIN + Skv, left, 3),
        ]
        rdmas = []
        for buf, src_row, dst_row, tgt, i in plan:
            r = pltpu.make_async_remote_copy(
                src_ref=buf.at[:, pl.ds(src_row, WIN), :],
                dst_ref=buf.at[:, pl.ds(dst_row, WIN), :],
                send_sem=send_sems.at[i],
                recv_sem=recv_sems.at[i],
                device_id=(tgt,),
                device_id_type=pl.DeviceIdType.MESH,
            )
            r.start()
            rdmas.append(r)

        wq16 = (wq_ref[...] * 0.125).astype(jnp.bfloat16)
        qall = jnp.dot(
            x_ref[...].reshape(B * Sq, Dm).astype(jnp.bfloat16), wq16,
            preferred_element_type=jnp.float32,
        ).astype(jnp.bfloat16)

        biases = []
        for qb in range(NQB):
            qi = lax.broadcasted_iota(jnp.int32, (WIN, KB), 0) + qb * WIN
            kj = lax.broadcasted_iota(jnp.int32, (WIN, KB), 1) + qb * WIN
            diff = kj - qi
            kg = me * Skv - WIN + kj
            mask = (diff >= 0) & (diff <= 2 * WIN) & (kg >= 0) & (kg < N_DEV * Skv)
            biases.append(jnp.where(mask, 0.0, -1e9).astype(jnp.float32))

        def attn_block(b, qb):
            koff = qb * WIN
            ctxs = []
            for h in range(Hq):
                q = qall[b * Sq + koff:b * Sq + koff + WIN,
                         h * Dh:(h + 1) * Dh]
                kh = kbuf[b, koff:koff + KB, h * Dh:(h + 1) * Dh]
                vh = vbuf[b, koff:koff + KB, h * Dh:(h + 1) * Dh]
                s = lax.dot_general(
                    q, kh, (((1,), (1,)), ((), ())),
                    preferred_element_type=jnp.float32,
                ) + biases[qb]
                w = jnp.exp(s)
                wsum = jnp.sum(w, axis=-1, keepdims=True)
                c = jnp.dot(w.astype(jnp.bfloat16), vh,
                            preferred_element_type=jnp.float32)
                ctxs.append(c / wsum)
            return jnp.concatenate(ctxs, axis=1)

        rdmas[0].wait()
        rdmas[1].wait()
        ctx_blocks = [[None] * NQB for _ in range(B)]
        for b in range(B):
            ctx_blocks[b][0] = attn_block(b, 0)
        rdmas[2].wait()
        rdmas[3].wait()
        for b in range(B):
            for qb in range(1, NQB):
                ctx_blocks[b][qb] = attn_block(b, qb)

        ctx = jnp.concatenate(
            [blk for row in ctx_blocks for blk in row], axis=0
        ).astype(jnp.bfloat16)
        out = jnp.dot(ctx, wo_ref[...].astype(jnp.bfloat16),
                      preferred_element_type=jnp.float32)
        out_ref[...] = out.reshape(B, Sq, Dm)

    return pl.pallas_call(
        body,
        out_shape=jax.ShapeDtypeStruct((B, Sq, Dm), jnp.float32),
        in_specs=[pl.BlockSpec(memory_space=pltpu.VMEM)] * 5,
        out_specs=pl.BlockSpec(memory_space=pltpu.VMEM),
        scratch_shapes=[
            pltpu.VMEM((B, Se, D), jnp.bfloat16),
            pltpu.VMEM((B, Se, D), jnp.bfloat16),
            pltpu.SemaphoreType.DMA((4,)),
            pltpu.SemaphoreType.DMA((4,)),
        ],
        compiler_params=pltpu.CompilerParams(collective_id=0),
    )(x, Wq, K_ext, V_ext, Wo)
